# Initial kernel scaffold; baseline (speedup 1.0000x reference)
import jax
import jax.numpy as jnp
from jax.experimental import pallas as pl


def kernel(patches, qstart):
    raise NotImplementedError("write your pallas kernel here")


# speedup vs baseline: 124.5040x; 124.5040x over previous
import functools

import jax
import jax.numpy as jnp
from jax import lax
from jax.experimental import pallas as pl
from jax.experimental.pallas import tpu as pltpu
from jax.experimental.pallas import tpu_sc as plsc

_T, _N, _C, _PS = 2, 65536, 3, 7
_H, _W = 256, 256
_HW = _H * _W
_SW = _W - _PS + 1
_PP = _PS * _PS
_NCOMBO = _T * _C
_SUBS = 5
_NWORK = _NCOMBO * _SUBS
_CHUNK = 256
_NCHUNKS = _N // _CHUNK
_C0 = (0, 52, 103, 154, 205)


_CW = _CHUNK * _PP


def _sc_body(marsh_hbm, qstart_hbm, part_hbm, canvas, pbuf, qbuf, bbuf):
    cid = lax.axis_index("c")
    sid = lax.axis_index("s")
    wid = sid * 2 + cid

    zero16 = jnp.zeros((16,), jnp.float32)

    def zbody(i, carry):
        canvas[pl.ds(i * 16, 16)] = zero16
        return carry

    lax.fori_loop(0, _HW // 16, zbody, 0)

    @pl.when(wid < _NWORK)
    def _work():
        plane = wid // _SUBS
        sub = wid % _SUBS

        iota = lax.iota(jnp.int32, 16)
        seven = jnp.full((16,), 7, jnp.int32)
        swv = jnp.full((16,), _SW, jnp.int32)
        offv = []
        for k in range(4):
            o = iota + (k * 16)
            offv.append(lax.div(o, seven) * _W + lax.rem(o, seven))
        m3 = (iota + 48) < _PP

        c0 = jnp.int32(0)
        for s in range(1, _SUBS):
            c0 = jnp.where(sub == s, jnp.int32(_C0[s]), c0)
        c1 = jnp.int32(_NCHUNKS)
        for s in range(_SUBS - 1):
            c1 = jnp.where(sub == s, jnp.int32(_C0[s + 1]), c1)

        def chunk_body(cix, carry):
            n0 = cix * _CHUNK
            pltpu.sync_copy(qstart_hbm.at[pl.ds(n0, _CHUNK)], qbuf)
            pltpu.sync_copy(
                marsh_hbm.at[plane, pl.ds(cix * _CW, _CW)],
                pbuf.at[pl.ds(0, _CW)],
            )

            def bbody(i, carry2):
                qv = qbuf[pl.ds(i * 16, 16)]
                hi = lax.div(qv, swv)
                bbuf[pl.ds(i * 16, 16)] = qv + 6 * hi
                return carry2

            lax.fori_loop(0, _CHUNK // 16, bbody, 0)

            def pbody(j, carry2):
                jv = jnp.full((16,), j, jnp.int32)
                bspl = plsc.load_gather(bbuf, [jv])
                pv = jnp.full((16,), j * _PP, jnp.int32) + iota
                for k in range(3):
                    vals = plsc.load_gather(pbuf, [pv + (k * 16)])
                    plsc.addupdate_scatter(canvas, [bspl + offv[k]], vals)
                vals = plsc.load_gather(pbuf, [pv + 48], mask=m3)
                plsc.addupdate_scatter(canvas, [bspl + offv[3]], vals, mask=m3)
                return carry2

            lax.fori_loop(0, _CHUNK, pbody, 0)
            return carry

        lax.fori_loop(c0, c1, chunk_body, 0)
        pltpu.sync_copy(canvas, part_hbm.at[wid])


_scatter_sc = functools.partial(
    pl.kernel,
    out_type=jax.ShapeDtypeStruct((_NWORK, _HW), jnp.float32),
    mesh=plsc.VectorSubcoreMesh(core_axis_name="c", subcore_axis_name="s"),
    compiler_params=pltpu.CompilerParams(needs_layout_passes=False),
    scratch_types=[
        pltpu.VMEM((_HW,), jnp.float32),
        pltpu.VMEM((_CW + 64,), jnp.float32),
        pltpu.VMEM((_CHUNK,), jnp.int32),
        pltpu.VMEM((_CHUNK,), jnp.int32),
    ],
)(_sc_body)


def _reduce_body(p_ref, o_ref):
    o_ref[...] = jnp.sum(p_ref[...], axis=1)


def kernel(patches, qstart):
    marsh = jnp.transpose(
        patches.reshape(_T, _N, _C, _PP), (0, 2, 1, 3)
    ).reshape(_NCOMBO, _N * _PP)
    partials = _scatter_sc(marsh, qstart)
    p4 = partials.reshape(_NCOMBO, _SUBS, _HW // 128, 128)
    vid6 = pl.pallas_call(
        _reduce_body,
        grid=(_NCOMBO,),
        in_specs=[
            pl.BlockSpec((1, _SUBS, _HW // 128, 128), lambda i: (i, 0, 0, 0))
        ],
        out_specs=pl.BlockSpec((1, _HW // 128, 128), lambda i: (i, 0, 0)),
        out_shape=jax.ShapeDtypeStruct((_NCOMBO, _HW // 128, 128), jnp.float32),
    )(p4)
    return vid6.reshape(_T, _C, _H, _W)


# speedup vs baseline: 332.8220x; 2.6732x over previous
import functools

import jax
import jax.numpy as jnp
from jax import lax
from jax.experimental import pallas as pl
from jax.experimental.pallas import tpu as pltpu
from jax.experimental.pallas import tpu_sc as plsc

_T, _N, _C, _PS = 2, 65536, 3, 7
_H, _W = 256, 256
_HW = _H * _W
_SW = _W - _PS + 1
_PP = _PS * _PS
_NCOMBO = _T * _C
_SUBS = 5
_NWORK = _NCOMBO * _SUBS
_CHUNK = 128
_NCHUNKS = _N // _CHUNK
_C0 = (0, 103, 206, 308, 410)


_ROW = _C * _PP


def _sc_body(patches_hbm, qstart_hbm, part_hbm, canvas, pbuf, qbuf, bbuf):
    cid = lax.axis_index("c")
    sid = lax.axis_index("s")
    wid = sid * 2 + cid

    zero16 = jnp.zeros((16,), jnp.float32)

    def zbody(i, carry):
        canvas[pl.ds(i * 16, 16)] = zero16
        return carry

    lax.fori_loop(0, _HW // 16, zbody, 0)

    @pl.when(wid < _NWORK)
    def _work():
        plane = wid // _SUBS
        sub = wid % _SUBS
        t = plane // _C
        ch = plane % _C

        iota = lax.iota(jnp.int32, 16)
        colb = jnp.full((16,), ch * _PP, jnp.int32) + iota
        seven = jnp.full((16,), 7, jnp.int32)
        swv = jnp.full((16,), _SW, jnp.int32)
        offv = []
        for k in range(4):
            o = iota + (k * 16)
            offv.append(lax.div(o, seven) * _W + lax.rem(o, seven))
        m3 = (iota + 48) < _PP

        c0 = jnp.int32(0)
        for s in range(1, _SUBS):
            c0 = jnp.where(sub == s, jnp.int32(_C0[s]), c0)
        c1 = jnp.int32(_NCHUNKS)
        for s in range(_SUBS - 1):
            c1 = jnp.where(sub == s, jnp.int32(_C0[s + 1]), c1)

        def chunk_body(cix, carry):
            n0 = cix * _CHUNK
            pltpu.sync_copy(qstart_hbm.at[pl.ds(n0, _CHUNK)], qbuf)
            pltpu.sync_copy(
                patches_hbm.at[t, pl.ds(n0, _CHUNK), 0, :],
                pbuf.at[pl.ds(0, _CHUNK)],
            )

            def bbody(i, carry2):
                qv = qbuf[pl.ds(i * 16, 16)]
                hi = lax.div(qv, swv)
                bbuf[pl.ds(i * 16, 16)] = qv + 6 * hi
                return carry2

            lax.fori_loop(0, _CHUNK // 16, bbody, 0)

            def pbody(j, carry2):
                jv = jnp.full((16,), j, jnp.int32)
                bspl = plsc.load_gather(bbuf, [jv])
                for k in range(3):
                    vals = plsc.load_gather(pbuf, [jv, colb + (k * 16)])
                    plsc.addupdate_scatter(canvas, [bspl + offv[k]], vals)
                vals = plsc.load_gather(pbuf, [jv, colb + 48], mask=m3)
                plsc.addupdate_scatter(canvas, [bspl + offv[3]], vals, mask=m3)
                return carry2

            lax.fori_loop(0, _CHUNK, pbody, 0)
            return carry

        lax.fori_loop(c0, c1, chunk_body, 0)
        pltpu.sync_copy(canvas, part_hbm.at[wid])


_scatter_sc = functools.partial(
    pl.kernel,
    out_type=jax.ShapeDtypeStruct((_NWORK, _HW), jnp.float32),
    mesh=plsc.VectorSubcoreMesh(core_axis_name="c", subcore_axis_name="s"),
    compiler_params=pltpu.CompilerParams(needs_layout_passes=False),
    scratch_types=[
        pltpu.VMEM((_HW,), jnp.float32),
        pltpu.VMEM((_CHUNK + 1, _ROW), jnp.float32),
        pltpu.VMEM((_CHUNK,), jnp.int32),
        pltpu.VMEM((_CHUNK,), jnp.int32),
    ],
)(_sc_body)


def _reduce_body(p_ref, o_ref):
    o_ref[...] = jnp.sum(p_ref[...], axis=1)


def kernel(patches, qstart):
    partials = _scatter_sc(patches, qstart)
    p4 = partials.reshape(_NCOMBO, _SUBS, _HW // 128, 128)
    vid6 = pl.pallas_call(
        _reduce_body,
        grid=(_NCOMBO,),
        in_specs=[
            pl.BlockSpec((1, _SUBS, _HW // 128, 128), lambda i: (i, 0, 0, 0))
        ],
        out_specs=pl.BlockSpec((1, _HW // 128, 128), lambda i: (i, 0, 0)),
        out_shape=jax.ShapeDtypeStruct((_NCOMBO, _HW // 128, 128), jnp.float32),
    )(p4)
    return vid6.reshape(_T, _C, _H, _W)


# speedup vs baseline: 447.2314x; 1.3438x over previous
import functools

import jax
import jax.numpy as jnp
from jax import lax
from jax.experimental import pallas as pl
from jax.experimental.pallas import tpu as pltpu
from jax.experimental.pallas import tpu_sc as plsc

_T, _N, _C, _PS = 2, 65536, 3, 7
_H, _W = 256, 256
_HW = _H * _W
_SW = _W - _PS + 1
_PP = _PS * _PS
_ROW = _C * _PP
_NCOMBO = _T * _C
_SUBS = 5
_NWORK = _NCOMBO * _SUBS
_CHUNK = 64
_NCHUNKS = _N // _CHUNK
_C0 = (0, 205, 410, 615, 820)
_QB = 205 * _CHUNK
_UNROLL = 8


def _sc_body(patches_hbm, qstart_hbm, part_hbm, canvas, pbufa, pbufb, qbuf,
             sema, semb):
    cid = lax.axis_index("c")
    sid = lax.axis_index("s")
    wid = sid * 2 + cid

    zero16 = jnp.zeros((16,), jnp.float32)

    def zbody(i, carry):
        canvas[pl.ds(i * 16, 16)] = zero16
        return carry

    lax.fori_loop(0, _HW // 16, zbody, 0)

    @pl.when(wid < _NWORK)
    def _work():
        plane = wid // _SUBS
        sub = wid % _SUBS
        t = plane // _C
        ch = plane % _C

        iota = lax.iota(jnp.int32, 16)
        colb = jnp.full((16,), ch * _PP, jnp.int32) + iota
        seven = jnp.full((16,), 7, jnp.int32)
        offv = []
        for k in range(4):
            o = iota + (k * 16)
            offv.append(lax.div(o, seven) * _W + lax.rem(o, seven))
        m3 = (iota + 48) < _PP

        c0 = jnp.int32(0)
        for s in range(1, _SUBS):
            c0 = jnp.where(sub == s, jnp.int32(_C0[s]), c0)
        c1 = jnp.int32(_NCHUNKS)
        for s in range(_SUBS - 1):
            c1 = jnp.where(sub == s, jnp.int32(_C0[s + 1]), c1)

        s0 = jnp.minimum(c0 * _CHUNK, jnp.int32(_N - _QB))
        pltpu.sync_copy(qstart_hbm.at[pl.ds(s0, _QB)], qbuf)

        def bbody(i, carry):
            qv = qbuf[pl.ds(i * 16, 16)]
            qu = qv.astype(jnp.uint32)
            hi = ((qu * jnp.uint32(67109)) >> jnp.uint32(24)).astype(jnp.int32)
            qbuf[pl.ds(i * 16, 16)] = qv + 6 * hi
            return carry

        lax.fori_loop(0, _QB // 16, bbody, 0)

        def dma(cix, buf, sem):
            return pltpu.make_async_copy(
                patches_hbm.at[t, pl.ds(cix * _CHUNK, _CHUNK), 0, :],
                buf.at[pl.ds(0, _CHUNK)],
                sem,
            )

        def compute(cix, buf):
            base_off = cix * _CHUNK - s0

            def pgroup(jg, carry):
                for u in range(_UNROLL):
                    j = jg * _UNROLL + u
                    jv = jnp.full((16,), j, jnp.int32)
                    bspl = plsc.load_gather(qbuf, [jv + base_off])
                    for k in range(3):
                        vals = plsc.load_gather(buf, [jv, colb + (k * 16)])
                        plsc.addupdate_scatter(canvas, [bspl + offv[k]], vals)
                    vals = plsc.load_gather(buf, [jv, colb + 48], mask=m3)
                    plsc.addupdate_scatter(
                        canvas, [bspl + offv[3]], vals, mask=m3
                    )
                return carry

            lax.fori_loop(0, _CHUNK // _UNROLL, pgroup, 0)

        dma(c0, pbufa, sema).start()

        def pair_body(i2, carry):
            ca = c0 + 2 * i2
            cb = ca + 1
            dma(ca, pbufa, sema).wait()

            @pl.when(cb < c1)
            def _():
                dma(cb, pbufb, semb).start()

            compute(ca, pbufa)

            @pl.when(cb < c1)
            def _():
                dma(cb, pbufb, semb).wait()

                @pl.when(cb + 1 < c1)
                def _():
                    dma(cb + 1, pbufa, sema).start()

                compute(cb, pbufb)

            return carry

        npairs = (c1 - c0 + 1) // 2
        lax.fori_loop(0, npairs, pair_body, 0)
        pltpu.sync_copy(canvas, part_hbm.at[wid])


_scatter_sc = functools.partial(
    pl.kernel,
    out_type=jax.ShapeDtypeStruct((_NWORK, _HW), jnp.float32),
    mesh=plsc.VectorSubcoreMesh(core_axis_name="c", subcore_axis_name="s"),
    compiler_params=pltpu.CompilerParams(needs_layout_passes=False),
    scratch_types=[
        pltpu.VMEM((_HW,), jnp.float32),
        pltpu.VMEM((_CHUNK + 1, _ROW), jnp.float32),
        pltpu.VMEM((_CHUNK + 1, _ROW), jnp.float32),
        pltpu.VMEM((_QB,), jnp.int32),
        pltpu.SemaphoreType.DMA,
        pltpu.SemaphoreType.DMA,
    ],
)(_sc_body)


def _reduce_body(p_ref, o_ref):
    o_ref[...] = jnp.sum(p_ref[...], axis=1)


def kernel(patches, qstart):
    partials = _scatter_sc(patches, qstart)
    p4 = partials.reshape(_NCOMBO, _SUBS, _HW // 128, 128)
    vid6 = pl.pallas_call(
        _reduce_body,
        grid=(_NCOMBO,),
        in_specs=[
            pl.BlockSpec((1, _SUBS, _HW // 128, 128), lambda i: (i, 0, 0, 0))
        ],
        out_specs=pl.BlockSpec((1, _HW // 128, 128), lambda i: (i, 0, 0)),
        out_shape=jax.ShapeDtypeStruct((_NCOMBO, _HW // 128, 128), jnp.float32),
    )(p4)
    return vid6.reshape(_T, _C, _H, _W)


# speedup vs baseline: 465.1833x; 1.0401x over previous
import functools

import jax
import jax.numpy as jnp
from jax import lax
from jax.experimental import pallas as pl
from jax.experimental.pallas import tpu as pltpu
from jax.experimental.pallas import tpu_sc as plsc

_T, _N, _C, _PS = 2, 65536, 3, 7
_H, _W = 256, 256
_HW = _H * _W
_SW = _W - _PS + 1
_PP = _PS * _PS
_ROW = _C * _PP
_NCOMBO = _T * _C
_SUBS = 5
_NWORK = _NCOMBO * _SUBS
_CHUNK = 64
_NCHUNKS = _N // _CHUNK
_C0 = (0, 205, 410, 615, 820)
_QB = 205 * _CHUNK
_UNROLL = 8


def _sc_body(patches_hbm, qstart_hbm, part_hbm, canvas, pbufa, pbufb, qbuf,
             sema, semb):
    cid = lax.axis_index("c")
    sid = lax.axis_index("s")
    wid = sid * 2 + cid

    zero16 = jnp.zeros((16,), jnp.float32)

    @plsc.parallel_loop(0, _HW, step=16, unroll=8)
    def _zero(i):
        canvas[pl.ds(i, 16)] = zero16

    @pl.when(wid < _NWORK)
    def _work():
        plane = wid // _SUBS
        sub = wid % _SUBS
        t = plane // _C
        ch = plane % _C

        iota = lax.iota(jnp.int32, 16)
        colb = jnp.full((16,), ch * _PP, jnp.int32) + iota
        seven = jnp.full((16,), 7, jnp.int32)
        offv = []
        for k in range(4):
            o = iota + (k * 16)
            offv.append(lax.div(o, seven) * _W + lax.rem(o, seven))
        m3 = (iota + 48) < _PP

        c0 = jnp.int32(0)
        for s in range(1, _SUBS):
            c0 = jnp.where(sub == s, jnp.int32(_C0[s]), c0)
        c1 = jnp.int32(_NCHUNKS)
        for s in range(_SUBS - 1):
            c1 = jnp.where(sub == s, jnp.int32(_C0[s + 1]), c1)

        s0 = jnp.minimum(c0 * _CHUNK, jnp.int32(_N - _QB))
        pltpu.sync_copy(qstart_hbm.at[pl.ds(s0, _QB)], qbuf)

        @plsc.parallel_loop(0, _QB, step=16, unroll=8)
        def _bases(i):
            qv = qbuf[pl.ds(i, 16)]
            qu = qv.astype(jnp.uint32)
            hi = ((qu * jnp.uint32(67109)) >> jnp.uint32(24)).astype(jnp.int32)
            qbuf[pl.ds(i, 16)] = qv + 6 * hi

        def dma(cix, buf, sem):
            return pltpu.make_async_copy(
                patches_hbm.at[t, pl.ds(cix * _CHUNK, _CHUNK), 0, :],
                buf.at[pl.ds(0, _CHUNK)],
                sem,
            )

        def compute(cix, buf):
            base_off = cix * _CHUNK - s0

            @plsc.parallel_loop(0, _CHUNK, step=1, unroll=_UNROLL)
            def _patches(j):
                jv = jnp.full((16,), j, jnp.int32)
                bspl = plsc.load_gather(qbuf, [jv + base_off])
                for k in range(3):
                    vals = plsc.load_gather(buf, [jv, colb + (k * 16)])
                    plsc.addupdate_scatter(canvas, [bspl + offv[k]], vals)
                vals = plsc.load_gather(buf, [jv, colb + 48], mask=m3)
                plsc.addupdate_scatter(canvas, [bspl + offv[3]], vals, mask=m3)

        dma(c0, pbufa, sema).start()

        def pair_body(i2, carry):
            ca = c0 + 2 * i2
            cb = ca + 1
            dma(ca, pbufa, sema).wait()

            @pl.when(cb < c1)
            def _():
                dma(cb, pbufb, semb).start()

            compute(ca, pbufa)

            @pl.when(cb < c1)
            def _():
                dma(cb, pbufb, semb).wait()

                @pl.when(cb + 1 < c1)
                def _():
                    dma(cb + 1, pbufa, sema).start()

                compute(cb, pbufb)

            return carry

        npairs = (c1 - c0 + 1) // 2
        lax.fori_loop(0, npairs, pair_body, 0)
        pltpu.sync_copy(canvas, part_hbm.at[wid])


_scatter_sc = functools.partial(
    pl.kernel,
    out_type=jax.ShapeDtypeStruct((_NWORK, _HW), jnp.float32),
    mesh=plsc.VectorSubcoreMesh(core_axis_name="c", subcore_axis_name="s"),
    compiler_params=pltpu.CompilerParams(needs_layout_passes=False),
    scratch_types=[
        pltpu.VMEM((_HW,), jnp.float32),
        pltpu.VMEM((_CHUNK + 1, _ROW), jnp.float32),
        pltpu.VMEM((_CHUNK + 1, _ROW), jnp.float32),
        pltpu.VMEM((_QB,), jnp.int32),
        pltpu.SemaphoreType.DMA,
        pltpu.SemaphoreType.DMA,
    ],
)(_sc_body)


def _reduce_body(p_ref, o_ref):
    o_ref[...] = jnp.sum(p_ref[...], axis=1)


def kernel(patches, qstart):
    partials = _scatter_sc(patches, qstart)
    p4 = partials.reshape(_NCOMBO, _SUBS, _HW // 128, 128)
    vid6 = pl.pallas_call(
        _reduce_body,
        grid=(_NCOMBO,),
        in_specs=[
            pl.BlockSpec((1, _SUBS, _HW // 128, 128), lambda i: (i, 0, 0, 0))
        ],
        out_specs=pl.BlockSpec((1, _HW // 128, 128), lambda i: (i, 0, 0)),
        out_shape=jax.ShapeDtypeStruct((_NCOMBO, _HW // 128, 128), jnp.float32),
    )(p4)
    return vid6.reshape(_T, _C, _H, _W)


# speedup vs baseline: 465.7654x; 1.0013x over previous
import functools

import jax
import jax.numpy as jnp
from jax import lax
from jax.experimental import pallas as pl
from jax.experimental.pallas import tpu as pltpu
from jax.experimental.pallas import tpu_sc as plsc

_T, _N, _C, _PS = 2, 65536, 3, 7
_H, _W = 256, 256
_HW = _H * _W
_SW = _W - _PS + 1
_PP = _PS * _PS
_ROW = _C * _PP
_NCOMBO = _T * _C
_SUBS = 5
_NWORK = _NCOMBO * _SUBS
_CHUNK = 64
_NCHUNKS = _N // _CHUNK
_C0 = (0, 205, 410, 615, 820)
_QB = 205 * _CHUNK
_UNROLL = 8


def _sc_body(patches_hbm, qstart_hbm, part_hbm, canvas, pbufa, pbufb, qbuf,
             sema, semb):
    cid = lax.axis_index("c")
    sid = lax.axis_index("s")
    wid = sid * 2 + cid

    zero16 = jnp.zeros((16,), jnp.float32)

    @plsc.parallel_loop(0, _HW, step=16, unroll=8)
    def _zero(i):
        canvas[pl.ds(i, 16)] = zero16

    @pl.when(wid < _NWORK)
    def _work():
        plane = wid // _SUBS
        sub = wid % _SUBS
        t = plane // _C
        ch = plane % _C

        iota = lax.iota(jnp.int32, 16)
        colb = jnp.full((16,), ch * _PP, jnp.int32) + iota
        seven = jnp.full((16,), 7, jnp.int32)
        offv = []
        for k in range(4):
            o = iota + (k * 16)
            offv.append(lax.div(o, seven) * _W + lax.rem(o, seven))
        m3 = (iota + 48) < _PP

        c0 = jnp.int32(0)
        for s in range(1, _SUBS):
            c0 = jnp.where(sub == s, jnp.int32(_C0[s]), c0)
        c1 = jnp.int32(_NCHUNKS)
        for s in range(_SUBS - 1):
            c1 = jnp.where(sub == s, jnp.int32(_C0[s + 1]), c1)

        s0 = jnp.minimum(c0 * _CHUNK, jnp.int32(_N - _QB))
        pltpu.sync_copy(qstart_hbm.at[pl.ds(s0, _QB)], qbuf)

        @plsc.parallel_loop(0, _QB, step=16, unroll=8)
        def _bases(i):
            qv = qbuf[pl.ds(i, 16)]
            qu = qv.astype(jnp.uint32)
            hi = ((qu * jnp.uint32(67109)) >> jnp.uint32(24)).astype(jnp.int32)
            qbuf[pl.ds(i, 16)] = qv + 6 * hi

        def dma(cix, buf, sem):
            return pltpu.make_async_copy(
                patches_hbm.at[t, pl.ds(cix * _CHUNK, _CHUNK), 0, :],
                buf.at[pl.ds(0, _CHUNK)],
                sem,
            )

        def compute(cix, buf):
            base_off = cix * _CHUNK - s0

            @plsc.parallel_loop(0, _CHUNK, step=1, unroll=_UNROLL)
            def _patches(j):
                jv = jnp.full((16,), j, jnp.int32)
                bspl = plsc.load_gather(qbuf, [jv + base_off])
                for k in range(3):
                    vals = plsc.load_gather(buf, [jv, colb + (k * 16)])
                    plsc.addupdate_scatter(canvas, [bspl + offv[k]], vals)
                vals = plsc.load_gather(buf, [jv, colb + 48], mask=m3)
                plsc.addupdate_scatter(canvas, [bspl + offv[3]], vals, mask=m3)

        dma(c0, pbufa, sema).start()

        def pair_body(i2, carry):
            ca = c0 + 2 * i2
            cb = ca + 1
            dma(ca, pbufa, sema).wait()

            @pl.when(cb < c1)
            def _():
                dma(cb, pbufb, semb).start()

            compute(ca, pbufa)

            @pl.when(cb < c1)
            def _():
                dma(cb, pbufb, semb).wait()

                @pl.when(cb + 1 < c1)
                def _():
                    dma(cb + 1, pbufa, sema).start()

                compute(cb, pbufb)

            return carry

        npairs = (c1 - c0 + 1) // 2
        lax.fori_loop(0, npairs, pair_body, 0)
        pltpu.sync_copy(canvas, part_hbm.at[wid])


_scatter_sc = functools.partial(
    pl.kernel,
    out_type=jax.ShapeDtypeStruct((_NWORK, _HW), jnp.float32),
    mesh=plsc.VectorSubcoreMesh(core_axis_name="c", subcore_axis_name="s"),
    compiler_params=pltpu.CompilerParams(
        needs_layout_passes=False, use_tc_tiling_on_sc=True
    ),
    scratch_types=[
        pltpu.VMEM((_HW,), jnp.float32),
        pltpu.VMEM((_CHUNK + 1, _ROW), jnp.float32),
        pltpu.VMEM((_CHUNK + 1, _ROW), jnp.float32),
        pltpu.VMEM((_QB,), jnp.int32),
        pltpu.SemaphoreType.DMA,
        pltpu.SemaphoreType.DMA,
    ],
)(_sc_body)


def _reduce_body(p_ref, o_ref):
    o_ref[...] = jnp.sum(p_ref[...], axis=1)


def kernel(patches, qstart):
    partials = _scatter_sc(patches, qstart)
    p4 = partials.reshape(_NCOMBO, _SUBS, _HW // 128, 128)
    vid6 = pl.pallas_call(
        _reduce_body,
        grid=(_NCOMBO,),
        in_specs=[
            pl.BlockSpec((1, _SUBS, _HW // 128, 128), lambda i: (i, 0, 0, 0))
        ],
        out_specs=pl.BlockSpec((1, _HW // 128, 128), lambda i: (i, 0, 0)),
        out_shape=jax.ShapeDtypeStruct((_NCOMBO, _HW // 128, 128), jnp.float32),
    )(p4)
    return vid6.reshape(_T, _C, _H, _W)


# speedup vs baseline: 621.9846x; 1.3354x over previous
import functools

import jax
import jax.numpy as jnp
from jax import lax
from jax.experimental import pallas as pl
from jax.experimental.pallas import tpu as pltpu
from jax.experimental.pallas import tpu_sc as plsc

_T, _N, _C, _PS = 2, 65536, 3, 7
_H, _W = 256, 256
_HW = _H * _W
_SW = _W - _PS + 1
_PP = _PS * _PS
_ROW = _C * _PP
_NCOMBO = _T * _C
_SUBS = 5
_NWORK = _NCOMBO * _SUBS
_CHUNK = 256
_NCHUNKS = _N // _CHUNK
_C0 = (0, 52, 103, 154, 205)
_QB = 52 * _CHUNK
_UNROLL = 8


def _sc_body(patches_hbm, qstart_hbm, part_hbm, canvas, pbufa, pbufb, qbuf,
             sema, semb):
    cid = lax.axis_index("c")
    sid = lax.axis_index("s")
    wid = sid * 2 + cid

    zero16 = jnp.zeros((16,), jnp.float32)

    @plsc.parallel_loop(0, _HW, step=16, unroll=8)
    def _zero(i):
        canvas[pl.ds(i, 16)] = zero16

    @pl.when(wid < _NWORK)
    def _work():
        plane = wid // _SUBS
        sub = wid % _SUBS
        t = plane // _C
        ch = plane % _C

        iota = lax.iota(jnp.int32, 16)
        seven = jnp.full((16,), 7, jnp.int32)
        offv = []
        for k in range(4):
            o = iota + (k * 16)
            offv.append(lax.div(o, seven) * _W + lax.rem(o, seven))
        m3 = (iota + 48) < _PP

        c0 = jnp.int32(0)
        for s in range(1, _SUBS):
            c0 = jnp.where(sub == s, jnp.int32(_C0[s]), c0)
        c1 = jnp.int32(_NCHUNKS)
        for s in range(_SUBS - 1):
            c1 = jnp.where(sub == s, jnp.int32(_C0[s + 1]), c1)

        s0 = jnp.minimum(c0 * _CHUNK, jnp.int32(_N - _QB))
        pltpu.sync_copy(qstart_hbm.at[pl.ds(s0, _QB)], qbuf)

        @plsc.parallel_loop(0, _QB, step=16, unroll=8)
        def _bases(i):
            qv = qbuf[pl.ds(i, 16)]
            qu = qv.astype(jnp.uint32)
            hi = ((qu * jnp.uint32(67109)) >> jnp.uint32(24)).astype(jnp.int32)
            qbuf[pl.ds(i, 16)] = qv + 6 * hi

        def dma(cix, buf, sem):
            return pltpu.make_async_copy(
                patches_hbm.at[
                    t, pl.ds(ch * _PP, _PP), 0, pl.ds(cix * _CHUNK, _CHUNK)
                ],
                buf,
                sem,
            )

        def compute(cix, buf):
            base_off = cix * _CHUNK - s0

            @plsc.parallel_loop(0, _CHUNK, step=1, unroll=_UNROLL)
            def _patches(j):
                jv = jnp.full((16,), j, jnp.int32)
                bspl = plsc.load_gather(qbuf, [jv + base_off])
                for k in range(3):
                    vals = plsc.load_gather(buf, [iota + (k * 16), jv])
                    plsc.addupdate_scatter(canvas, [bspl + offv[k]], vals)
                vals = plsc.load_gather(buf, [iota + 48, jv], mask=m3)
                plsc.addupdate_scatter(canvas, [bspl + offv[3]], vals, mask=m3)

        dma(c0, pbufa, sema).start()

        def pair_body(i2, carry):
            ca = c0 + 2 * i2
            cb = ca + 1
            dma(ca, pbufa, sema).wait()

            @pl.when(cb < c1)
            def _():
                dma(cb, pbufb, semb).start()

            compute(ca, pbufa)

            @pl.when(cb < c1)
            def _():
                dma(cb, pbufb, semb).wait()

                @pl.when(cb + 1 < c1)
                def _():
                    dma(cb + 1, pbufa, sema).start()

                compute(cb, pbufb)

            return carry

        npairs = (c1 - c0 + 1) // 2
        lax.fori_loop(0, npairs, pair_body, 0)
        pltpu.sync_copy(canvas, part_hbm.at[wid])


_scatter_sc = functools.partial(
    pl.kernel,
    out_type=jax.ShapeDtypeStruct((_NWORK, _HW), jnp.float32),
    mesh=plsc.VectorSubcoreMesh(core_axis_name="c", subcore_axis_name="s"),
    compiler_params=pltpu.CompilerParams(needs_layout_passes=False),
    scratch_types=[
        pltpu.VMEM((_HW,), jnp.float32),
        pltpu.VMEM((_PP, _CHUNK), jnp.float32),
        pltpu.VMEM((_PP, _CHUNK), jnp.float32),
        pltpu.VMEM((_QB,), jnp.int32),
        pltpu.SemaphoreType.DMA,
        pltpu.SemaphoreType.DMA,
    ],
)(_sc_body)


def _reduce_body(p_ref, o_ref):
    o_ref[...] = jnp.sum(p_ref[...], axis=1)


def kernel(patches, qstart):
    pt = jnp.transpose(patches.reshape(_T, _N, _ROW), (0, 2, 1))
    partials = _scatter_sc(pt.reshape(_T, _ROW, 1, _N), qstart)
    p4 = partials.reshape(_NCOMBO, _SUBS, _HW // 128, 128)
    vid6 = pl.pallas_call(
        _reduce_body,
        grid=(_NCOMBO,),
        in_specs=[
            pl.BlockSpec((1, _SUBS, _HW // 128, 128), lambda i: (i, 0, 0, 0))
        ],
        out_specs=pl.BlockSpec((1, _HW // 128, 128), lambda i: (i, 0, 0)),
        out_shape=jax.ShapeDtypeStruct((_NCOMBO, _HW // 128, 128), jnp.float32),
    )(p4)
    return vid6.reshape(_T, _C, _H, _W)
